# baseline (speedup 1.0000x reference)
import jax
import jax.numpy as jnp
from jax.experimental import pallas as pl
from jax.experimental.pallas import tpu as pltpu

_CHUNKS = (512, 1024, 2048, 4096, 512)


def _dma_pipe(emb_ref, out_ref, *args):
    n = len(_CHUNKS)
    bufs = args[:n]
    in_sems, out_sems = args[n], args[n + 1]
    offs = [sum(_CHUNKS[:i]) for i in range(n)]

    def in_copy(i):
        return pltpu.make_async_copy(
            emb_ref.at[pl.ds(offs[i], _CHUNKS[i]), :], bufs[i], in_sems.at[i]
        )

    def out_copy(i):
        return pltpu.make_async_copy(
            bufs[i], out_ref.at[pl.ds(offs[i], _CHUNKS[i]), :], out_sems.at[i]
        )

    for i in range(n):
        in_copy(i).start()
    for i in range(n):
        in_copy(i).wait()
        out_copy(i).start()
    for i in range(n):
        out_copy(i).wait()


def kernel(x, emb):
    T = x.shape[1]
    D = emb.shape[1]
    assert sum(_CHUNKS) == T
    n = len(_CHUNKS)
    out = pl.pallas_call(
        _dma_pipe,
        in_specs=[pl.BlockSpec(memory_space=pl.ANY)],
        out_specs=pl.BlockSpec(memory_space=pl.ANY),
        out_shape=jax.ShapeDtypeStruct((T, D), emb.dtype),
        scratch_shapes=[pltpu.VMEM((c, D), emb.dtype) for c in _CHUNKS]
        + [
            pltpu.SemaphoreType.DMA((n,)),
            pltpu.SemaphoreType.DMA((n,)),
        ],
    )(emb[:T])
    return out[None, :, :]


# speedup vs baseline: 1.0113x; 1.0113x over previous
import jax
import jax.numpy as jnp
from jax.experimental import pallas as pl
from jax.experimental.pallas import tpu as pltpu

_BLOCK = 4096


def _copy_block(emb_ref, out_ref):
    out_ref[:, :] = emb_ref[:, :]


def kernel(x, emb):
    T = x.shape[1]
    D = emb.shape[1]
    assert T % _BLOCK == 0
    out = pl.pallas_call(
        _copy_block,
        grid=(T // _BLOCK,),
        in_specs=[pl.BlockSpec((_BLOCK, D), lambda i: (i, 0))],
        out_specs=pl.BlockSpec((_BLOCK, D), lambda i: (i, 0)),
        out_shape=jax.ShapeDtypeStruct((T, D), emb.dtype),
        compiler_params=pltpu.CompilerParams(
            dimension_semantics=("parallel",),
        ),
    )(emb[:T])
    return out[None, :, :]
